# baseline (speedup 1.0000x reference)
import jax
import jax.numpy as jnp
from jax import lax
from jax.experimental import pallas as pl
from jax.experimental.pallas import tpu as pltpu

_CP = pltpu.CompilerParams(
    dimension_semantics=("arbitrary",),
    vmem_limit_bytes=56 * 1024 * 1024,
)

_CP2 = pltpu.CompilerParams(
    dimension_semantics=("arbitrary", "arbitrary"),
    vmem_limit_bytes=56 * 1024 * 1024,
)

_F32 = jnp.float32


def _bspec(shape, batch_split):
    if batch_split:
        blk = (1,) + tuple(shape[1:])
        nd = len(shape)
        return pl.BlockSpec(blk, lambda b: (b,) + (0,) * (nd - 1))
    return pl.BlockSpec(tuple(shape), lambda b: (0,) * len(shape))


def _call(body, B, ins, split_flags, out_shapes):
    in_specs = [_bspec(a.shape, f) for a, f in zip(ins, split_flags)]
    multi = isinstance(out_shapes, (list, tuple))
    outs = out_shapes if multi else [out_shapes]
    out_specs = [_bspec(s.shape, True) for s in outs]
    return pl.pallas_call(
        body,
        grid=(B,),
        in_specs=in_specs,
        out_specs=out_specs if multi else out_specs[0],
        out_shape=out_shapes,
        compiler_params=_CP,
    )(*ins)


def _c4s2_body(xs_ref, w_ref, b_ref, o_ref):
    _, Ho, Wo, Co = o_ref.shape
    C4 = xs_ref.shape[3]
    acc = None
    for da in range(2):
        for db in range(2):
            p = xs_ref[pl.ds(0, 1), pl.ds(da, Ho), pl.ds(db, Wo), :]
            t = jnp.dot(p.reshape(Ho * Wo, C4), w_ref[da, db],
                        preferred_element_type=_F32)
            acc = t if acc is None else acc + t
    o = jnp.maximum(acc + b_ref[...], 0.0)
    o_ref[...] = o.reshape(1, Ho, Wo, Co)


def _conv4s2_relu(x, w, b):
    B, H, W, Ci = x.shape
    Co = w.shape[3]
    Ho, Wo = H // 2, W // 2
    xp = jnp.pad(x, ((0, 0), (1, 1), (1, 1), (0, 0)))
    xs = (xp.reshape(B, Ho + 1, 2, Wo + 1, 2, Ci)
          .transpose(0, 1, 3, 2, 4, 5)
          .reshape(B, Ho + 1, Wo + 1, 4 * Ci))
    w4 = (w.reshape(2, 2, 2, 2, Ci, Co)
          .transpose(0, 2, 1, 3, 4, 5)
          .reshape(2, 2, 4 * Ci, Co))
    out = jax.ShapeDtypeStruct((B, Ho, Wo, Co), _F32)
    return _call(_c4s2_body, B, (xs, w4, b.reshape(1, Co)),
                 (True, False, False), out)


def _res_body(xp_ref, w3_ref, b3_ref, w1_ref, b1_ref, o_ref):
    _, H, W, C = o_ref.shape
    acc = None
    for dy in range(3):
        for dx in range(3):
            p = jnp.maximum(xp_ref[0, dy:dy + H, dx:dx + W, :], 0.0)
            t = jnp.dot(p.reshape(H * W, C), w3_ref[dy, dx],
                        preferred_element_type=_F32)
            acc = t if acc is None else acc + t
    h = jnp.maximum(acc + b3_ref[...], 0.0)
    h = jnp.dot(h, w1_ref[...], preferred_element_type=_F32) + b1_ref[...]
    o = xp_ref[0, 1:1 + H, 1:1 + W, :].reshape(H * W, C) + h
    o_ref[...] = o.reshape(1, H, W, C)


def _res_block(x, w3, b3, w1, b1):
    B, H, W, C = x.shape
    xp = jnp.pad(x, ((0, 0), (1, 1), (1, 1), (0, 0)))
    out = jax.ShapeDtypeStruct((B, H, W, C), _F32)
    return _call(_res_body, B, (xp, w3, b3.reshape(1, C), w1, b1.reshape(1, C)),
                 (True, False, False, False, False), out)


def _vq_body(h_ref, w_ref, b_ref, emb_ref, e2_ref, z_ref, zq_ref, idx_ref):
    _, H, W, C = h_ref.shape
    NE, E = emb_ref.shape
    hm = h_ref[0].reshape(H * W, C)
    z = jnp.dot(hm, w_ref[...], preferred_element_type=_F32) + b_ref[...]
    z2 = jnp.sum(z * z, axis=1, keepdims=True)
    two_dot = 2.0 * lax.dot_general(z, emb_ref[...], (((1,), (1,)), ((), ())),
                                    preferred_element_type=_F32)
    d = (z2 + e2_ref[...]) - two_dot
    dmin = jnp.min(d, axis=1, keepdims=True)
    iota = lax.broadcasted_iota(jnp.int32, (H * W, NE), 1)
    idxk = jnp.min(jnp.where(d == dmin, iota, NE), axis=1, keepdims=True)
    oh = (iota == idxk).astype(_F32)
    zq = lax.dot_general(oh, emb_ref[...], (((1,), (0,)), ((), ())),
                         preferred_element_type=_F32)
    z_ref[...] = z.reshape(1, H, W, E)
    zq_ref[...] = zq.reshape(1, H * W, E)
    idx_ref[...] = jnp.broadcast_to(idxk, (H * W, 128)).reshape(1, H * W, 128)


def _vq(h, weo, eob, emb):
    B, H, W, C = h.shape
    E = weo.shape[1]
    e2 = jnp.sum(emb * emb, axis=1).reshape(1, -1)
    outs = (
        jax.ShapeDtypeStruct((B, H, W, E), _F32),
        jax.ShapeDtypeStruct((B, H * W, E), _F32),
        jax.ShapeDtypeStruct((B, H * W, 128), jnp.int32),
    )
    return _call(_vq_body, B, (h, weo, eob.reshape(1, E), emb, e2),
                 (True, False, False, False, False), outs)


def _d0_body(zp_ref, z_ref, w_ref, b_ref, o_ref, l_ref):
    _, H, W, E = z_ref.shape
    Co = o_ref.shape[3]
    acc = None
    for dy in range(3):
        for dx in range(3):
            p = zp_ref[0, dy:dy + H, dx:dx + W, :]
            t = jnp.dot(p.reshape(H * W, E), w_ref[dy, dx],
                        preferred_element_type=_F32)
            acc = t if acc is None else acc + t
    o_ref[...] = (acc + b_ref[...]).reshape(1, H, W, Co)
    diff = zp_ref[0, 1:1 + H, 1:1 + W, :] - z_ref[0]
    s = jnp.sum(diff * diff)
    l_ref[...] = jnp.full((1, 1, 128), s, _F32)


def _d0(zq_nhwc, z, w, b):
    B, H, W, E = z.shape
    Co = w.shape[3]
    zp = jnp.pad(zq_nhwc, ((0, 0), (1, 1), (1, 1), (0, 0)))
    outs = (
        jax.ShapeDtypeStruct((B, H, W, Co), _F32),
        jax.ShapeDtypeStruct((B, 1, 128), _F32),
    )
    return _call(_d0_body, B, (zp, z, w, b.reshape(1, Co)),
                 (True, True, False, False), outs)


def _ct_body(xb_ref, w_ref, b_ref, o_ref, *, act):
    _, Hb, W, C4 = o_ref.shape
    Ci = xb_ref.shape[4]
    acc = None
    for ay in range(3):
        for ax in range(3):
            p = xb_ref[pl.ds(0, 1), pl.ds(0, 1), pl.ds(ay, Hb), pl.ds(ax, W), :]
            t = jnp.dot(p.reshape(Hb * W, Ci), w_ref[ay, ax],
                        preferred_element_type=_F32)
            acc = t if acc is None else acc + t
    o = act(acc + b_ref[...])
    o_ref[...] = o.reshape(1, Hb, W, C4)


def _ct_relu_body(xp_ref, w_ref, b_ref, o_ref):
    _ct_body(xp_ref, w_ref, b_ref, o_ref, act=lambda v: jnp.maximum(v, 0.0))


def _ct_tanh_body(xp_ref, w_ref, b_ref, o_ref):
    _ct_body(xp_ref, w_ref, b_ref, o_ref, act=jnp.tanh)


def _convt(x, w, b, body, nb=1):
    B, H, W, Ci = x.shape
    Co = w.shape[3]
    Hb = H // nb
    xp = jnp.pad(x, ((0, 0), (1, 1), (1, 1), (0, 0)))
    xb = jnp.stack([xp[:, Hb * r:Hb * r + Hb + 2] for r in range(nb)], axis=1)
    wb = jnp.zeros((3, 3, Ci, 2, 2, Co), w.dtype)
    for ay in range(3):
        for r in range(2):
            ky = r - 2 * ay + 3
            if not 0 <= ky <= 3:
                continue
            for ax in range(3):
                for s in range(2):
                    kx = s - 2 * ax + 3
                    if not 0 <= kx <= 3:
                        continue
                    wb = wb.at[ay, ax, :, r, s, :].set(w[ky, kx])
    wb = wb.reshape(3, 3, Ci, 4 * Co)
    bt = jnp.tile(b, 4).reshape(1, 4 * Co)
    packed = pl.pallas_call(
        body,
        grid=(B, nb),
        in_specs=[
            pl.BlockSpec((1, 1, Hb + 2, W + 2, Ci), lambda b, r: (b, r, 0, 0, 0)),
            pl.BlockSpec(wb.shape, lambda b, r: (0, 0, 0, 0)),
            pl.BlockSpec((1, 4 * Co), lambda b, r: (0, 0)),
        ],
        out_specs=pl.BlockSpec((1, Hb, W, 4 * Co), lambda b, r: (b, r, 0, 0)),
        out_shape=jax.ShapeDtypeStruct((B, H, W, 4 * Co), _F32),
        compiler_params=_CP2,
    )(xb, wb, bt)
    return (packed.reshape(B, H, W, 2, 2, Co)
            .transpose(0, 1, 3, 2, 4, 5)
            .reshape(B, 2 * H, 2 * W, Co))


def kernel(x, ew1, eb1, ew2, eb2, er1w3, er1b3, er1w1, er1b1, er2w3, er2b3,
           er2w1, er2b1, eow, eob, emb, dw0, db0, dr1w3, dr1b3, dr1w1, dr1b1,
           dr2w3, dr2b3, dr2w1, dr2b1, dt1w, dt1b, dt2w, dt2b):
    B = x.shape[0]
    HL = x.shape[2] // 4
    E = emb.shape[1]

    xn = jnp.transpose(x, (0, 2, 3, 1))
    h = _conv4s2_relu(xn, jnp.transpose(ew1, (2, 3, 1, 0)), eb1)
    h = _conv4s2_relu(h, jnp.transpose(ew2, (2, 3, 1, 0)), eb2)
    h = _res_block(h, jnp.transpose(er1w3, (2, 3, 1, 0)), er1b3,
                   er1w1[:, :, 0, 0].T, er1b1)
    h = _res_block(h, jnp.transpose(er2w3, (2, 3, 1, 0)), er2b3,
                   er2w1[:, :, 0, 0].T, er2b1)

    z, zq_rows, idx = _vq(h, eow[:, :, 0, 0].T, eob, emb)
    zq_nchw = zq_rows.reshape(B, E, HL, HL)
    zq_nhwc = jnp.transpose(zq_nchw, (0, 2, 3, 1))

    g, lparts = _d0(zq_nhwc, z, jnp.transpose(dw0, (2, 3, 1, 0)), db0)
    loss_vq = 1.25 * jnp.sum(lparts[:, 0, 0]) / (B * HL * HL * E)
    g = _res_block(g, jnp.transpose(dr1w3, (2, 3, 1, 0)), dr1b3,
                   dr1w1[:, :, 0, 0].T, dr1b1)
    g = _res_block(g, jnp.transpose(dr2w3, (2, 3, 1, 0)), dr2b3,
                   dr2w1[:, :, 0, 0].T, dr2b1)
    g = _convt(g, jnp.transpose(dt1w, (2, 3, 0, 1)), dt1b, _ct_relu_body)
    xr = _convt(g, jnp.transpose(dt2w, (2, 3, 0, 1)), dt2b, _ct_tanh_body, nb=4)

    x_recon = jnp.transpose(xr, (0, 3, 1, 2))
    return x_recon, loss_vq, idx[:, :, 0].reshape(-1)[:, None]
